# baseline (speedup 1.0000x reference)
import functools

import jax
import jax.numpy as jnp
from jax import lax
from jax.experimental import pallas as pl
from jax.experimental.pallas import tpu as pltpu
from jax.experimental.pallas import tpu_sc as plsc

_TABLE_ROWS = 1_000_000
_D = 64
_DP = 128
_B = 819_200

_info = plsc.get_sparse_core_info()
_NC, _NS = _info.num_cores, _info.num_subcores
_NW = _NC * _NS
_BPW = _B // _NW
_CH = 256
_NCHUNK = _BPW // _CH

_mesh = plsc.VectorSubcoreMesh(core_axis_name="c", subcore_axis_name="s")


@functools.partial(
    pl.kernel,
    out_type=jax.ShapeDtypeStruct((_D, _B), jnp.float32),
    mesh=_mesh,
    scratch_types=[
        pltpu.VMEM((_BPW,), jnp.int32),
        pltpu.VMEM((2, _CH, _DP), jnp.float32),
        pltpu.SemaphoreType.DMA,
        pltpu.SemaphoreType.DMA,
        pltpu.SemaphoreType.DMA,
        pltpu.SemaphoreType.DMA,
    ],
)
def _gather(table_hbm, idx_hbm, out_t_hbm, idx_v, rows_v, gsem0, gsem1, wsem0, wsem1):
    wid = lax.axis_index("s") * _NC + lax.axis_index("c")
    base = wid * _BPW
    gsems = (gsem0, gsem1)
    wsems = (wsem0, wsem1)

    pltpu.sync_copy(idx_hbm.at[pl.ds(base, _BPW)], idx_v)

    def fire_gather(g, b):
        return pltpu.async_copy(
            table_hbm.at[idx_v.at[pl.ds(g * _CH, _CH)]], rows_v.at[b], gsems[b]
        )

    def wait_gather(b):
        pltpu.make_async_copy(
            table_hbm.at[pl.ds(0, _CH)], rows_v.at[b], gsems[b]
        ).wait()

    def write_chunk(g, b):
        off = pl.multiple_of(base + g * _CH, _CH)
        handles = [
            pltpu.async_copy(
                rows_v.at[b, :, f], out_t_hbm.at[f, pl.ds(off, _CH)], wsems[b]
            )
            for f in range(_D)
        ]
        for h in handles:
            h.wait()

    fire_gather(0, 0)
    fire_gather(1, 1)

    def pair_body(p):
        for b in range(2):
            g = 2 * p + b
            wait_gather(b)
            write_chunk(g, b)
            fire_gather(g + 2, b)

    pl.loop(0, _NCHUNK // 2 - 1)(pair_body)

    for b in range(2):
        wait_gather(b)
        write_chunk(_NCHUNK - 2 + b, b)


@jax.jit
def kernel(input, index, _):
    tpad = jnp.pad(input, ((0, 0), (0, _DP - _D)))
    out_t = _gather(tpad, index.astype(jnp.int32))
    gathered = out_t.T
    return (input, index, gathered)


# speedup vs baseline: 90.8589x; 90.8589x over previous
import functools

import jax
import jax.numpy as jnp
from jax import lax
from jax.experimental import pallas as pl
from jax.experimental.pallas import tpu as pltpu
from jax.experimental.pallas import tpu_sc as plsc

_TABLE_ROWS = 1_000_000
_D = 64
_DP = 128
_B = 819_200

_info = plsc.get_sparse_core_info()
_NC, _NS = _info.num_cores, _info.num_subcores
_NW = _NC * _NS
_BPW = _B // _NW
_CH = 400
_NCHUNK = _BPW // _CH

_PAD_BLK = 8000

_mesh = plsc.VectorSubcoreMesh(core_axis_name="c", subcore_axis_name="s")


def _pad_body(in_ref, out_ref):
    out_ref[:, : _D] = in_ref[...]


_pad_copy = pl.pallas_call(
    _pad_body,
    grid=(_TABLE_ROWS // _PAD_BLK,),
    in_specs=[pl.BlockSpec((_PAD_BLK, _D), lambda i: (i, 0))],
    out_specs=pl.BlockSpec((_PAD_BLK, _DP), lambda i: (i, 0)),
    out_shape=jax.ShapeDtypeStruct((_TABLE_ROWS, _DP), jnp.float32),
)


@functools.partial(
    pl.kernel,
    out_type=jax.ShapeDtypeStruct((_B, _DP), jnp.float32),
    mesh=_mesh,
    scratch_types=[
        pltpu.VMEM((_BPW,), jnp.int32),
        pltpu.VMEM((2, _CH, _DP), jnp.float32),
        pltpu.SemaphoreType.DMA,
        pltpu.SemaphoreType.DMA,
        pltpu.SemaphoreType.DMA,
        pltpu.SemaphoreType.DMA,
    ],
)
def _gather(table_hbm, idx_hbm, out_hbm, idx_v, rows_v, gsem0, gsem1, wsem0, wsem1):
    wid = lax.axis_index("s") * _NC + lax.axis_index("c")
    base = wid * _BPW
    gsems = (gsem0, gsem1)
    wsems = (wsem0, wsem1)

    pltpu.sync_copy(idx_hbm.at[pl.ds(base, _BPW)], idx_v)

    gathers = [None, None]
    writes = [None, None]
    for g in range(2):
        gathers[g % 2] = pltpu.async_copy(
            table_hbm.at[idx_v.at[pl.ds(g * _CH, _CH)]], rows_v.at[g % 2], gsems[g % 2]
        )

    for g in range(_NCHUNK):
        b = g % 2
        gathers[b].wait()
        writes[b] = pltpu.async_copy(
            rows_v.at[b], out_hbm.at[pl.ds(base + g * _CH, _CH)], wsems[b]
        )
        if g + 2 < _NCHUNK:
            writes[b].wait()
            gathers[b] = pltpu.async_copy(
                table_hbm.at[idx_v.at[pl.ds((g + 2) * _CH, _CH)]],
                rows_v.at[b],
                gsems[b],
            )
    writes[(_NCHUNK - 2) % 2].wait()
    writes[(_NCHUNK - 1) % 2].wait()


@jax.jit
def kernel(input, index, _):
    tpad = _pad_copy(input)
    padded_out = _gather(tpad, index.astype(jnp.int32))
    gathered = padded_out[:, :_D]
    return (input, index, gathered)
